# Initial kernel scaffold; baseline (speedup 1.0000x reference)
import jax
import jax.numpy as jnp
from jax.experimental import pallas as pl


def kernel(video_nodes, video_neighs_list, video_neighs_weights_list, video_embeddings, W1, b1, W2, b2, w3, b3):
    raise NotImplementedError("write your pallas kernel here")


# speedup vs baseline: 3.2020x; 3.2020x over previous
import functools

import jax
import jax.numpy as jnp
from jax import lax
from jax.experimental import pallas as pl
from jax.experimental.pallas import tpu as pltpu
from jax.experimental.pallas import tpu_sc as plsc

NC, NS = 2, 16
NW = NC * NS


def _sc_gather_body(b1w, b2w, ch, table_hbm, idxn_hbm, idxu_hbm,
                    outn_hbm, outu_hbm, idx_v, rows_v, idx2_v, rows2_v, sem):
    wid = lax.axis_index("s") * NC + lax.axis_index("c")
    base = wid * b1w

    def chunk(c, carry):
        off = base + c * ch
        pltpu.sync_copy(idxn_hbm.at[pl.ds(off, ch)], idx_v)
        pltpu.async_copy(table_hbm.at[idx_v], rows_v, sem).wait()
        pltpu.sync_copy(rows_v, outn_hbm.at[pl.ds(off, ch)])
        return carry

    lax.fori_loop(0, b1w // ch, chunk, 0)

    base2 = wid * b2w
    pltpu.sync_copy(idxu_hbm.at[pl.ds(base2, b2w)], idx2_v)
    pltpu.async_copy(table_hbm.at[idx2_v], rows2_v, sem).wait()
    pltpu.sync_copy(rows2_v, outu_hbm.at[pl.ds(base2, b2w)])


def _sc_gather(table, idx_neigh, idx_node, ch):
    b1, b2 = idx_neigh.shape[0], idx_node.shape[0]
    d = table.shape[1]
    b1w, b2w = b1 // NW, b2 // NW
    mesh = plsc.VectorSubcoreMesh(core_axis_name="c", subcore_axis_name="s")
    k = pl.kernel(
        functools.partial(_sc_gather_body, b1w, b2w, ch),
        out_type=(jax.ShapeDtypeStruct((b1, d), table.dtype),
                  jax.ShapeDtypeStruct((b2, d), table.dtype)),
        mesh=mesh,
        scratch_types=[
            pltpu.VMEM((ch,), jnp.int32),
            pltpu.VMEM((ch, d), table.dtype),
            pltpu.VMEM((b2w,), jnp.int32),
            pltpu.VMEM((b2w, d), table.dtype),
            pltpu.SemaphoreType.DMA,
        ],
    )
    return k(table, idx_neigh, idx_node)


def _tc_body(nb, deg, d, neigh_ref, node_ref, w1a_ref, w1b_ref, w2_ref,
             b1_ref, b2_ref, w3_ref, b3_ref, out_ref):
    neigh = neigh_ref[...]
    nodep = jnp.dot(node_ref[...], w1b_ref[...],
                    preferred_element_type=jnp.float32) + b1_ref[...]
    h = jnp.dot(neigh, w1a_ref[...], preferred_element_type=jnp.float32)
    h = h.reshape(nb, deg, d) + nodep[:, None, :]
    h = jnp.maximum(h, 0.0).reshape(nb * deg, d)
    h = jnp.dot(h, w2_ref[...], preferred_element_type=jnp.float32) + b2_ref[...]
    h = jnp.maximum(h, 0.0)
    s = jnp.sum((h * w3_ref[...]).reshape(nb, deg, d), axis=-1) + b3_ref[0, 0]
    m = jnp.max(s, axis=1, keepdims=True)
    e = jnp.exp(s - m)
    att = e / jnp.sum(e, axis=1, keepdims=True)
    out_ref[...] = jnp.sum(att[:, :, None] * neigh.reshape(nb, deg, d), axis=1)


def _tc_attention(neigh_rows, node_emb, w1a, w1b, w2, b1r, b2r, w3r, b3r, nb):
    n, d = node_emb.shape
    deg = neigh_rows.shape[0] // n
    grid = n // nb
    return pl.pallas_call(
        functools.partial(_tc_body, nb, deg, d),
        grid=(grid,),
        in_specs=[
            pl.BlockSpec((nb * deg, d), lambda i: (i, 0)),
            pl.BlockSpec((nb, d), lambda i: (i, 0)),
            pl.BlockSpec((d, d), lambda i: (0, 0)),
            pl.BlockSpec((d, d), lambda i: (0, 0)),
            pl.BlockSpec((d, d), lambda i: (0, 0)),
            pl.BlockSpec((1, d), lambda i: (0, 0)),
            pl.BlockSpec((1, d), lambda i: (0, 0)),
            pl.BlockSpec((1, d), lambda i: (0, 0)),
            pl.BlockSpec((1, 1), lambda i: (0, 0)),
        ],
        out_specs=pl.BlockSpec((nb, d), lambda i: (i, 0)),
        out_shape=jax.ShapeDtypeStruct((n, d), jnp.float32),
        compiler_params=pltpu.CompilerParams(
            dimension_semantics=("arbitrary",)),
    )(neigh_rows, node_emb, w1a, w1b, w2, b1r, b2r, w3r, b3r)


def kernel(video_nodes, video_neighs_list, video_neighs_weights_list,
           video_embeddings, W1, b1, W2, b2, w3, b3):
    n, deg = video_neighs_list.shape
    v, d = video_embeddings.shape

    idx_neigh = video_neighs_list.reshape(-1)
    align = 8 * NW
    b2_pad = ((n + align - 1) // align) * align
    idx_node = jnp.concatenate(
        [video_nodes, jnp.zeros((b2_pad - n,), jnp.int32)])

    neigh_rows, node_rows = _sc_gather(video_embeddings, idx_neigh,
                                       idx_node, ch=400)
    node_emb = node_rows[:n]

    w1a, w1b = W1[:d], W1[d:]
    return _tc_attention(neigh_rows, node_emb, w1a, w1b, W2,
                         b1.reshape(1, d), b2.reshape(1, d),
                         w3.reshape(1, d), b3.reshape(1, 1), nb=200)


# speedup vs baseline: 3.4595x; 1.0804x over previous
import functools

import jax
import jax.numpy as jnp
from jax import lax
from jax.experimental import pallas as pl
from jax.experimental.pallas import tpu as pltpu
from jax.experimental.pallas import tpu_sc as plsc

NC, NS = 2, 16
NW = NC * NS


def _sc_gather_body(b1w, b2w, ch, table_hbm, idxn_hbm, idxu_hbm,
                    outn_hbm, outu_hbm, idx_all, rows0, rows1,
                    idx2_v, rows2_v, gsem0, gsem1, wsem0, wsem1, nsem):
    wid = lax.axis_index("s") * NC + lax.axis_index("c")
    base = wid * b1w
    nchunks = b1w // ch
    bufs = (rows0, rows1)
    gsems = (gsem0, gsem1)
    wsems = (wsem0, wsem1)

    pltpu.sync_copy(idxn_hbm.at[pl.ds(base, b1w)], idx_all)

    def start_gather(c, b):
        pltpu.async_copy(table_hbm.at[idx_all.at[pl.ds(c * ch, ch)]],
                         bufs[b], gsems[b])

    def wait_gather(b):
        pltpu.make_async_copy(table_hbm.at[idx_all.at[pl.ds(0, ch)]],
                              bufs[b], gsems[b]).wait()

    def start_write(c, b):
        pltpu.async_copy(bufs[b], outn_hbm.at[pl.ds(base + c * ch, ch)],
                         wsems[b])

    def wait_write(b):
        pltpu.make_async_copy(bufs[b], outn_hbm.at[pl.ds(base, ch)],
                              wsems[b]).wait()

    start_gather(0, 0)
    start_gather(1, 1)

    def step(i, carry):
        for b in range(2):
            c = i * 2 + b
            wait_gather(b)
            start_write(c, b)
            wait_write(b)
            start_gather(c + 2, b)
        return carry

    lax.fori_loop(0, nchunks // 2 - 1, step, 0)

    base2 = wid * b2w
    pltpu.sync_copy(idxu_hbm.at[pl.ds(base2, b2w)], idx2_v)
    pltpu.async_copy(table_hbm.at[idx2_v], rows2_v, nsem)

    for b in range(2):
        c = nchunks - 2 + b
        wait_gather(b)
        start_write(c, b)
        wait_write(b)

    pltpu.make_async_copy(table_hbm.at[idx2_v], rows2_v, nsem).wait()
    pltpu.sync_copy(rows2_v, outu_hbm.at[pl.ds(base2, b2w)])


def _sc_gather(table, idx_neigh, idx_node, ch):
    b1, b2 = idx_neigh.shape[0], idx_node.shape[0]
    d = table.shape[1]
    b1w, b2w = b1 // NW, b2 // NW
    mesh = plsc.VectorSubcoreMesh(core_axis_name="c", subcore_axis_name="s")
    k = pl.kernel(
        functools.partial(_sc_gather_body, b1w, b2w, ch),
        out_type=(jax.ShapeDtypeStruct((b1, d), table.dtype),
                  jax.ShapeDtypeStruct((b2, d), table.dtype)),
        mesh=mesh,
        scratch_types=[
            pltpu.VMEM((b1w,), jnp.int32),
            pltpu.VMEM((ch, d), table.dtype),
            pltpu.VMEM((ch, d), table.dtype),
            pltpu.VMEM((b2w,), jnp.int32),
            pltpu.VMEM((b2w, d), table.dtype),
            pltpu.SemaphoreType.DMA,
            pltpu.SemaphoreType.DMA,
            pltpu.SemaphoreType.DMA,
            pltpu.SemaphoreType.DMA,
            pltpu.SemaphoreType.DMA,
        ],
    )
    return k(table, idx_neigh, idx_node)


def _tc_body(nb, deg, d, neigh_ref, node_ref, w1a_ref, w1b_ref, w2_ref,
             b1_ref, b2_ref, w3_ref, b3_ref, out_ref):
    neigh = neigh_ref[...]
    nodep = jnp.dot(node_ref[...], w1b_ref[...],
                    preferred_element_type=jnp.float32) + b1_ref[...]
    h = jnp.dot(neigh, w1a_ref[...], preferred_element_type=jnp.float32)
    h = h.reshape(nb, deg, d) + nodep[:, None, :]
    h = jnp.maximum(h, 0.0).reshape(nb * deg, d)
    h = jnp.dot(h, w2_ref[...], preferred_element_type=jnp.float32) + b2_ref[...]
    h = jnp.maximum(h, 0.0)
    s = jnp.sum((h * w3_ref[...]).reshape(nb, deg, d), axis=-1) + b3_ref[0, 0]
    m = jnp.max(s, axis=1, keepdims=True)
    e = jnp.exp(s - m)
    att = e / jnp.sum(e, axis=1, keepdims=True)
    out_ref[...] = jnp.sum(att[:, :, None] * neigh.reshape(nb, deg, d), axis=1)


def _tc_attention(neigh_rows, node_emb, w1a, w1b, w2, b1r, b2r, w3r, b3r, nb):
    n, d = node_emb.shape
    deg = neigh_rows.shape[0] // n
    grid = n // nb
    return pl.pallas_call(
        functools.partial(_tc_body, nb, deg, d),
        grid=(grid,),
        in_specs=[
            pl.BlockSpec((nb * deg, d), lambda i: (i, 0)),
            pl.BlockSpec((nb, d), lambda i: (i, 0)),
            pl.BlockSpec((d, d), lambda i: (0, 0)),
            pl.BlockSpec((d, d), lambda i: (0, 0)),
            pl.BlockSpec((d, d), lambda i: (0, 0)),
            pl.BlockSpec((1, d), lambda i: (0, 0)),
            pl.BlockSpec((1, d), lambda i: (0, 0)),
            pl.BlockSpec((1, d), lambda i: (0, 0)),
            pl.BlockSpec((1, 1), lambda i: (0, 0)),
        ],
        out_specs=pl.BlockSpec((nb, d), lambda i: (i, 0)),
        out_shape=jax.ShapeDtypeStruct((n, d), jnp.float32),
        compiler_params=pltpu.CompilerParams(
            dimension_semantics=("arbitrary",)),
    )(neigh_rows, node_emb, w1a, w1b, w2, b1r, b2r, w3r, b3r)


def kernel(video_nodes, video_neighs_list, video_neighs_weights_list,
           video_embeddings, W1, b1, W2, b2, w3, b3):
    n, deg = video_neighs_list.shape
    v, d = video_embeddings.shape

    idx_neigh = video_neighs_list.reshape(-1)
    align = 8 * NW
    b2_pad = ((n + align - 1) // align) * align
    idx_node = jnp.concatenate(
        [video_nodes, jnp.zeros((b2_pad - n,), jnp.int32)])

    neigh_rows, node_rows = _sc_gather(video_embeddings, idx_neigh,
                                       idx_node, ch=200)
    node_emb = node_rows[:n]

    w1a, w1b = W1[:d], W1[d:]
    return _tc_attention(neigh_rows, node_emb, w1a, w1b, W2,
                         b1.reshape(1, d), b2.reshape(1, d),
                         w3.reshape(1, d), b3.reshape(1, 1), nb=200)


# speedup vs baseline: 3.5812x; 1.0352x over previous
import functools

import jax
import jax.numpy as jnp
from jax import lax
from jax.experimental import pallas as pl
from jax.experimental.pallas import tpu as pltpu
from jax.experimental.pallas import tpu_sc as plsc

NC, NS = 2, 16
NW = NC * NS


def _sc_gather_body(b1w, b2w, ch, table_hbm, idxn_hbm, idxu_hbm,
                    outn_hbm, outu_hbm, idx_all, rows0, rows1,
                    idx2_v, rows2_v, gsem0, gsem1, wsem0, wsem1, nsem):
    wid = lax.axis_index("s") * NC + lax.axis_index("c")
    base = wid * b1w
    nchunks = b1w // ch
    bufs = (rows0, rows1)
    gsems = (gsem0, gsem1)
    wsems = (wsem0, wsem1)

    pltpu.sync_copy(idxn_hbm.at[pl.ds(base, b1w)], idx_all)

    def start_gather(c, b):
        pltpu.async_copy(table_hbm.at[idx_all.at[pl.ds(c * ch, ch)]],
                         bufs[b], gsems[b])

    def wait_gather(b):
        pltpu.make_async_copy(table_hbm.at[idx_all.at[pl.ds(0, ch)]],
                              bufs[b], gsems[b]).wait()

    def start_write(c, b):
        pltpu.async_copy(bufs[b], outn_hbm.at[pl.ds(base + c * ch, ch)],
                         wsems[b])

    def wait_write(b):
        pltpu.make_async_copy(bufs[b], outn_hbm.at[pl.ds(base, ch)],
                              wsems[b]).wait()

    start_gather(0, 0)
    start_gather(1, 1)

    def step(i, carry):
        for b in range(2):
            c = i * 2 + b
            wait_gather(b)
            start_write(c, b)
            wait_write(b)
            start_gather(c + 2, b)
        return carry

    lax.fori_loop(0, nchunks // 2 - 1, step, 0)

    base2 = wid * b2w
    pltpu.sync_copy(idxu_hbm.at[pl.ds(base2, b2w)], idx2_v)
    pltpu.async_copy(table_hbm.at[idx2_v], rows2_v, nsem)

    for b in range(2):
        c = nchunks - 2 + b
        wait_gather(b)
        start_write(c, b)
        wait_write(b)

    pltpu.make_async_copy(table_hbm.at[idx2_v], rows2_v, nsem).wait()
    pltpu.sync_copy(rows2_v, outu_hbm.at[pl.ds(base2, b2w)])


def _sc_gather(table, idx_neigh, idx_node, ch):
    b1, b2 = idx_neigh.shape[0], idx_node.shape[0]
    d = table.shape[1]
    b1w, b2w = b1 // NW, b2 // NW
    mesh = plsc.VectorSubcoreMesh(core_axis_name="c", subcore_axis_name="s")
    k = pl.kernel(
        functools.partial(_sc_gather_body, b1w, b2w, ch),
        out_type=(jax.ShapeDtypeStruct((b1, d), table.dtype),
                  jax.ShapeDtypeStruct((b2, d), table.dtype)),
        mesh=mesh,
        scratch_types=[
            pltpu.VMEM((b1w,), jnp.int32),
            pltpu.VMEM((ch, d), table.dtype),
            pltpu.VMEM((ch, d), table.dtype),
            pltpu.VMEM((b2w,), jnp.int32),
            pltpu.VMEM((b2w, d), table.dtype),
            pltpu.SemaphoreType.DMA,
            pltpu.SemaphoreType.DMA,
            pltpu.SemaphoreType.DMA,
            pltpu.SemaphoreType.DMA,
            pltpu.SemaphoreType.DMA,
        ],
    )
    return k(table, idx_neigh, idx_node)


def _tc_body(nb, deg, d, neigh_ref, node_ref, w1a_ref, w1b_ref, w2_ref,
             b1_ref, b2_ref, w3rep_ref, out_ref, s_scr):
    neigh = neigh_ref[...]
    nodep = jnp.dot(node_ref[...], w1b_ref[...],
                    preferred_element_type=jnp.float32) + b1_ref[...]
    h = jnp.dot(neigh, w1a_ref[...], preferred_element_type=jnp.float32)
    h = h.reshape(nb, deg, d) + nodep[:, None, :]
    h = jnp.maximum(h, 0.0).reshape(nb * deg, d)
    h = jnp.dot(h, w2_ref[...], preferred_element_type=jnp.float32) + b2_ref[...]
    h = jnp.maximum(h, 0.0)
    smat = jnp.dot(h, w3rep_ref[...], preferred_element_type=jnp.float32)
    s_scr[...] = smat.reshape(nb, deg, d)[:, :, 0]
    sc = s_scr[...]
    m = jnp.max(sc, axis=1, keepdims=True)
    e = jnp.exp(sc - m)
    att = e / jnp.sum(e, axis=1, keepdims=True)
    out_ref[...] = jnp.sum(att[:, :, None] * neigh.reshape(nb, deg, d), axis=1)


def _tc_attention(neigh_rows, node_emb, w1a, w1b, w2, b1r, b2r, w3rep, nb):
    n, d = node_emb.shape
    deg = neigh_rows.shape[0] // n
    grid = n // nb
    return pl.pallas_call(
        functools.partial(_tc_body, nb, deg, d),
        grid=(grid,),
        in_specs=[
            pl.BlockSpec((nb * deg, d), lambda i: (i, 0)),
            pl.BlockSpec((nb, d), lambda i: (i, 0)),
            pl.BlockSpec((d, d), lambda i: (0, 0)),
            pl.BlockSpec((d, d), lambda i: (0, 0)),
            pl.BlockSpec((d, d), lambda i: (0, 0)),
            pl.BlockSpec((1, d), lambda i: (0, 0)),
            pl.BlockSpec((1, d), lambda i: (0, 0)),
            pl.BlockSpec((d, d), lambda i: (0, 0)),
        ],
        out_specs=pl.BlockSpec((nb, d), lambda i: (i, 0)),
        out_shape=jax.ShapeDtypeStruct((n, d), jnp.float32),
        scratch_shapes=[pltpu.VMEM((nb, deg), jnp.float32)],
        compiler_params=pltpu.CompilerParams(
            dimension_semantics=("arbitrary",)),
    )(neigh_rows, node_emb, w1a, w1b, w2, b1r, b2r, w3rep)


def kernel(video_nodes, video_neighs_list, video_neighs_weights_list,
           video_embeddings, W1, b1, W2, b2, w3, b3):
    n, deg = video_neighs_list.shape
    v, d = video_embeddings.shape

    idx_neigh = video_neighs_list.reshape(-1)
    align = 8 * NW
    b2_pad = ((n + align - 1) // align) * align
    idx_node = jnp.concatenate(
        [video_nodes, jnp.zeros((b2_pad - n,), jnp.int32)])

    neigh_rows, node_rows = _sc_gather(video_embeddings, idx_neigh,
                                       idx_node, ch=200)
    node_emb = node_rows[:n]

    w1a, w1b = W1[:d], W1[d:]
    w3rep = jnp.tile(w3, (1, d))
    return _tc_attention(neigh_rows, node_emb, w1a, w1b, W2,
                         b1.reshape(1, d), b2.reshape(1, d),
                         w3rep, nb=200)
